# baseline (speedup 1.0000x reference)
import jax
import jax.numpy as jnp
from jax.experimental import pallas as pl
from jax.experimental.pallas import tpu as pltpu

_B = 2048
_D = 128
_DA = _D + 8
_NCLS = 256
_MARGIN = 1.0
_BLK = 1024


def _dist(lhs, rhs):
    return jax.lax.dot_general(lhs, rhs, (((1,), (1,)), ((), ())),
                               preferred_element_type=jnp.float32)


def _triplet_kernel(e_ref, labc_ref, labr_ref, sum_ref, cnt_ref,
                    lhs_ref, rhs_ref, cmin_ref, fb_ref):
    i = pl.program_id(0)
    labr = labr_ref[...]
    labc = labc_ref[...]
    inf = jnp.float32(jnp.inf)

    @pl.when(i == 0)
    def _():
        e = e_ref[...]
        sq = jnp.sum(e * e, axis=1, keepdims=True)
        sqhi = sq.astype(jnp.bfloat16)
        sqlo = (sq - sqhi.astype(jnp.float32)).astype(jnp.bfloat16)
        one = jnp.ones((_B, 1), jnp.bfloat16)
        zp = jnp.zeros((_B, _DA - _D - 4), jnp.bfloat16)
        lhs_ref[:, 0:_D] = e.astype(jnp.bfloat16)
        lhs_ref[:, _D:_D + 1] = sqhi
        lhs_ref[:, _D + 1:_D + 2] = sqlo
        lhs_ref[:, _D + 2:_D + 3] = one
        lhs_ref[:, _D + 3:_D + 4] = one
        lhs_ref[:, _D + 4:] = zp
        rhs_ref[:, 0:_D] = (e * jnp.float32(-2.0)).astype(jnp.bfloat16)
        rhs_ref[:, _D:_D + 1] = one
        rhs_ref[:, _D + 1:_D + 2] = one
        rhs_ref[:, _D + 2:_D + 3] = sqhi
        rhs_ref[:, _D + 3:_D + 4] = sqlo
        rhs_ref[:, _D + 4:] = zp
        cls = jax.lax.broadcasted_iota(jnp.int32, (_NCLS, 1), 0)
        ohc = jnp.where(cls == labr, 1.0, 0.0)
        ncls = jnp.sum(ohc, axis=1, keepdims=True)
        s1 = jnp.sum(ncls * ncls, keepdims=True)
        cnt_ref[...] = (0.5 * (s1 - jnp.float32(_B))).astype(jnp.int32)

        dist = _dist(lhs_ref[0:_BLK, :], rhs_ref[...])
        eqm = labc == labr
        mh = jnp.where(eqm, inf, dist)
        neg = jnp.min(mh, axis=1, keepdims=True)
        cmin_ref[...] = jnp.min(mh[:, _BLK:], axis=0, keepdims=True)
        fb_ref[...] = dist[0:1, _BLK:]
        neg = jnp.where(neg < inf, neg, dist[:, 0:1])
        negt = neg - _MARGIN
        x = jnp.maximum(dist - negt, 0.0)
        colv = jax.lax.broadcasted_iota(jnp.int32, (1, _BLK), 1)
        rowv = jax.lax.broadcasted_iota(jnp.int32, (_BLK, 1), 0)
        posl = jnp.logical_and(eqm[:, 0:_BLK], colv > rowv)
        sl = jnp.sum(jnp.where(posl, x[:, 0:_BLK], 0.0), keepdims=True)
        su = jnp.sum(jnp.where(eqm[:, _BLK:], x[:, _BLK:], 0.0),
                     keepdims=True)
        sum_ref[...] = sl + su

    @pl.when(i == 1)
    def _():
        dist = _dist(lhs_ref[_BLK:, :], rhs_ref[_BLK:, :])
        labru = labr[:, _BLK:]
        eqm = labc == labru
        mh = jnp.where(eqm, inf, dist)
        neg = jnp.minimum(jnp.min(mh, axis=1, keepdims=True),
                          cmin_ref[...].T)
        neg = jnp.where(neg < inf, neg, fb_ref[...].T)
        negt = neg - _MARGIN
        x = jnp.maximum(dist - negt, 0.0)
        colv = jax.lax.broadcasted_iota(jnp.int32, (1, _BLK), 1)
        rowv = jax.lax.broadcasted_iota(jnp.int32, (_BLK, 1), 0)
        pos = jnp.logical_and(eqm, colv > rowv)
        total = sum_ref[...] + jnp.sum(jnp.where(pos, x, 0.0), keepdims=True)
        sum_ref[...] = total / cnt_ref[...].astype(jnp.float32)


def kernel(embeddings, target):
    labc = target.reshape(_B, 1)
    labr = target.reshape(1, _B)
    out_sum, out_cnt = pl.pallas_call(
        _triplet_kernel,
        grid=(2,),
        in_specs=[
            pl.BlockSpec((_B, _D), lambda i: (0, 0)),
            pl.BlockSpec((_BLK, 1), lambda i: (i, 0)),
            pl.BlockSpec((1, _B), lambda i: (0, 0)),
        ],
        out_specs=[
            pl.BlockSpec((1, 1), lambda i: (0, 0)),
            pl.BlockSpec((1, 1), lambda i: (0, 0)),
        ],
        out_shape=[
            jax.ShapeDtypeStruct((1, 1), jnp.float32),
            jax.ShapeDtypeStruct((1, 1), jnp.int32),
        ],
        scratch_shapes=[
            pltpu.VMEM((_B, _DA), jnp.bfloat16),
            pltpu.VMEM((_B, _DA), jnp.bfloat16),
            pltpu.VMEM((1, _BLK), jnp.float32),
            pltpu.VMEM((1, _BLK), jnp.float32),
        ],
    )(embeddings, labc, labr)
    return (out_sum[0, 0], out_cnt[0, 0])
